# baseline (speedup 1.0000x reference)
import jax
import jax.numpy as jnp
from jax.experimental import pallas as pl


def _sum_k_kernel(x_ref, o_ref):
    k = pl.program_id(1)

    @pl.when(k == 0)
    def _():
        o_ref[...] = x_ref[0]

    @pl.when(k > 0)
    def _():
        o_ref[...] += x_ref[0]


def kernel(encoded_edges, encoded_history):
    K, E, d = encoded_edges.shape
    BR = 8000
    return pl.pallas_call(
        _sum_k_kernel,
        grid=(E // BR, K),
        in_specs=[pl.BlockSpec((1, BR, d), lambda i, k: (k, i, 0))],
        out_specs=pl.BlockSpec((BR, d), lambda i, k: (i, 0)),
        out_shape=jax.ShapeDtypeStruct((E, d), encoded_edges.dtype),
    )(encoded_edges)
